# baseline (speedup 1.0000x reference)
import functools

import jax
import jax.numpy as jnp
from jax import lax
from jax.experimental import pallas as pl
from jax.experimental.pallas import tpu as pltpu
from jax.experimental.pallas import tpu_sc as plsc


def _mul_t_body(e_ref, w_ref, o_ref):
    o_ref[...] = (e_ref[...] * w_ref[...]).T


def _dense_mul_t(eT, wT):
    D_, V_ = eT.shape
    cols = 12800
    grid = (V_ + cols - 1) // cols
    in_spec = pl.BlockSpec((D_, cols), lambda i: (0, i))
    out_spec = pl.BlockSpec((cols, D_), lambda i: (i, 0))
    return pl.pallas_call(
        _mul_t_body,
        out_shape=jax.ShapeDtypeStruct((V_, D_), eT.dtype),
        grid=(grid,),
        in_specs=[in_spec, in_spec],
        out_specs=out_spec,
    )(eT, wT)


_CH = 128


def _make_gather(V, D, Bt, S):
    info = plsc.get_sparse_core_info()
    NC, NS = info.num_cores, info.num_subcores
    NW = NC * NS
    CH = _CH
    assert Bt == NW * CH and D % 16 == 0

    mesh = plsc.VectorSubcoreMesh(core_axis_name="c", subcore_axis_name="s")

    @functools.partial(
        pl.kernel, mesh=mesh,
        out_type=jax.ShapeDtypeStruct((S, D, Bt), jnp.float32),
        compiler_params=pltpu.CompilerParams(
            use_tc_tiling_on_sc=False, needs_layout_passes=False),
        scratch_types=[
            pltpu.VMEM((S, CH), jnp.int32),
            pltpu.VMEM((CH, D), jnp.float32),
            pltpu.VMEM((CH, D), jnp.float32),
            pltpu.VMEM((D, CH), jnp.float32),
            pltpu.VMEM((D, CH), jnp.float32),
            pltpu.SemaphoreType.DMA,
            pltpu.SemaphoreType.DMA,
            pltpu.SemaphoreType.DMA,
            pltpu.SemaphoreType.DMA,
        ],
    )
    def gather(table_hbm, idsb_hbm, out_hbm, idx_v, buf_a, buf_b, tb_a, tb_b,
               gsem_a, gsem_b, wsem_a, wsem_b):
        wid = lax.axis_index("s") * NC + lax.axis_index("c")
        pltpu.sync_copy(idsb_hbm.at[wid], idx_v)
        io16 = lax.iota(jnp.int32, 16)

        def wait_gather(buf_, gsem_):
            pltpu.make_async_copy(
                table_hbm.at[pl.ds(0, CH)], buf_, gsem_).wait()

        def wait_write(tb_, wsem_):
            pltpu.make_async_copy(
                tb_, out_hbm.at[0, pl.ds(0, D), pl.ds(0, CH)], wsem_).wait()

        pltpu.async_copy(table_hbm.at[idx_v.at[0]], buf_a, gsem_a)

        def step(k, buf, tb, gsem, wsem, obuf, ogsem):
            @pl.when(k + 1 < S)
            def _():
                pltpu.async_copy(table_hbm.at[idx_v.at[k + 1]], obuf, ogsem)

            wait_gather(buf, gsem)

            @pl.when(k >= 2)
            def _():
                wait_write(tb, wsem)

            def tr_body(d, _):
                cidx = jnp.full((16,), d, jnp.int32)
                for g in range(CH // 16):
                    v = plsc.load_gather(buf, [g * 16 + io16, cidx])
                    tb[d, pl.ds(g * 16, 16)] = v
                return 0

            lax.fori_loop(0, D, tr_body, 0, unroll=8)

            pltpu.async_copy(
                tb, out_hbm.at[k, pl.ds(0, D), pl.ds(wid * CH, CH)], wsem)

        def body(k, _):
            @pl.when(k % 2 == 0)
            def _():
                step(k, buf_a, tb_a, gsem_a, wsem_a, buf_b, gsem_b)

            @pl.when(k % 2 == 1)
            def _():
                step(k, buf_b, tb_b, gsem_b, wsem_b, buf_a, gsem_a)

            return 0

        lax.fori_loop(0, S, body, 0)
        wait_write(tb_a, wsem_a)
        wait_write(tb_b, wsem_b)

    return gather, NW


def kernel(inputs, embeddings, w):
    Bt, S = inputs.shape
    V, D = embeddings.shape
    emb = _dense_mul_t(embeddings.T, w.T)
    gather, NW = _make_gather(V, D, Bt, S)
    idsb = inputs.T.reshape(S, NW, _CH).transpose(1, 0, 2).astype(jnp.int32)
    outT = gather(emb, idsb)
    return outT.transpose(2, 0, 1)


# speedup vs baseline: 2.0069x; 2.0069x over previous
import functools

import jax
import jax.numpy as jnp
from jax import lax
from jax.experimental import pallas as pl
from jax.experimental.pallas import tpu as pltpu
from jax.experimental.pallas import tpu_sc as plsc


def _mul_t_body(e_ref, w_ref, o_ref):
    t = (e_ref[...] * w_ref[...]).T
    o_ref[:, : t.shape[1]] = t
    o_ref[:, t.shape[1]:] = jnp.zeros_like(t)


def _dense_mul_t(eT, wT):
    D_, V_ = eT.shape
    cols = 12800
    grid = (V_ + cols - 1) // cols
    in_spec = pl.BlockSpec((D_, cols), lambda i: (0, i))
    out_spec = pl.BlockSpec((cols, 2 * D_), lambda i: (i, 0))
    return pl.pallas_call(
        _mul_t_body,
        out_shape=jax.ShapeDtypeStruct((V_, 2 * D_), eT.dtype),
        grid=(grid,),
        in_specs=[in_spec, in_spec],
        out_specs=out_spec,
    )(eT, wT)


_CH = 128


def _make_gather(V2, D, B):
    info = plsc.get_sparse_core_info()
    NC, NS = info.num_cores, info.num_subcores
    NW = NC * NS
    bpw = B // NW
    CH = _CH
    nch = bpw // CH
    assert B % NW == 0 and bpw % CH == 0

    mesh = plsc.VectorSubcoreMesh(core_axis_name="c", subcore_axis_name="s")

    @functools.partial(
        pl.kernel, mesh=mesh,
        out_type=jax.ShapeDtypeStruct((B, D), jnp.float32),
        compiler_params=pltpu.CompilerParams(use_tc_tiling_on_sc=False),
        scratch_types=[
            pltpu.VMEM((nch, CH), jnp.int32),
            pltpu.VMEM((CH, D), jnp.float32),
            pltpu.VMEM((CH, D), jnp.float32),
            pltpu.SemaphoreType.DMA,
            pltpu.SemaphoreType.DMA,
            pltpu.SemaphoreType.DMA,
            pltpu.SemaphoreType.DMA,
        ],
    )
    def gather(table_hbm, idx_hbm, out_hbm, idx_v, buf_a, buf_b,
               gsem_a, gsem_b, wsem_a, wsem_b):
        wid = lax.axis_index("s") * NC + lax.axis_index("c")
        base = wid * bpw
        pltpu.sync_copy(idx_hbm.at[wid], idx_v)

        def wait_write(buf_, wsem_):
            pltpu.make_async_copy(
                buf_, out_hbm.at[pl.ds(base, CH)], wsem_).wait()

        def wait_gather(buf_, gsem_):
            pltpu.make_async_copy(
                table_hbm.at[pl.ds(0, CH)], buf_, gsem_).wait()

        pltpu.async_copy(table_hbm.at[idx_v.at[0]], buf_a, gsem_a)

        def step(k, buf, gsem, wsem, obuf, ogsem, owsem):
            @pl.when(k + 1 < nch)
            def _():
                @pl.when(k >= 1)
                def _():
                    wait_write(obuf, owsem)

                pltpu.async_copy(table_hbm.at[idx_v.at[k + 1]], obuf, ogsem)

            wait_gather(buf, gsem)
            pltpu.async_copy(buf, out_hbm.at[pl.ds(base + k * CH, CH)], wsem)

        def body(k, _):
            @pl.when(k % 2 == 0)
            def _():
                step(k, buf_a, gsem_a, wsem_a, buf_b, gsem_b, wsem_b)

            @pl.when(k % 2 == 1)
            def _():
                step(k, buf_b, gsem_b, wsem_b, buf_a, gsem_a, wsem_a)

            return 0

        lax.fori_loop(0, nch, body, 0)
        wait_write(buf_a, wsem_a)
        wait_write(buf_b, wsem_b)

    return gather


def kernel(inputs, embeddings, w):
    Bt, S = inputs.shape
    V, D = embeddings.shape
    B = Bt * S
    emb = _dense_mul_t(embeddings.T, w.T).reshape(2 * V, D)
    info = plsc.get_sparse_core_info()
    NW = info.num_cores * info.num_subcores
    ids3d = (inputs.reshape(NW, B // (NW * _CH), _CH) * 2).astype(jnp.int32)
    out = _make_gather(2 * V, D, B)(emb, ids3d)
    return out.reshape(Bt, S, D)


# speedup vs baseline: 3.0463x; 1.5179x over previous
import functools

import jax
import jax.numpy as jnp
from jax import lax
from jax.experimental import pallas as pl
from jax.experimental.pallas import tpu as pltpu
from jax.experimental.pallas import tpu_sc as plsc


def _mul_t_body(e_ref, w_ref, o_ref):
    t = (e_ref[...] * w_ref[...]).T
    o_ref[:, : t.shape[1]] = t
    o_ref[:, t.shape[1]:] = jnp.zeros_like(t)


def _dense_mul_t(eT, wT):
    D_, V_ = eT.shape
    cols = 12800
    grid = (V_ + cols - 1) // cols
    in_spec = pl.BlockSpec((D_, cols), lambda i: (0, i))
    out_spec = pl.BlockSpec((cols, 2 * D_), lambda i: (i, 0))
    return pl.pallas_call(
        _mul_t_body,
        out_shape=jax.ShapeDtypeStruct((V_, 2 * D_), eT.dtype),
        grid=(grid,),
        in_specs=[in_spec, in_spec],
        out_specs=out_spec,
    )(eT, wT)


_CH = 100
_SPAD = 56


def _make_gather(V2, D, B, Bt, S):
    info = plsc.get_sparse_core_info()
    NC, NS = info.num_cores, info.num_subcores
    NW = NC * NS
    bpw = B // NW
    CH = _CH
    nch = bpw // CH
    bblk = Bt // NW
    assert B % NW == 0 and bpw % CH == 0 and CH == 2 * S

    mesh = plsc.VectorSubcoreMesh(core_axis_name="c", subcore_axis_name="s")

    @functools.partial(
        pl.kernel, mesh=mesh,
        out_type=jax.ShapeDtypeStruct((Bt, _SPAD, 2 * D), jnp.float32),
        compiler_params=pltpu.CompilerParams(use_tc_tiling_on_sc=False),
        scratch_types=[
            pltpu.VMEM((nch, CH), jnp.int32),
            pltpu.VMEM((CH, D), jnp.float32),
            pltpu.VMEM((CH, D), jnp.float32),
            pltpu.SemaphoreType.DMA,
            pltpu.SemaphoreType.DMA,
            pltpu.SemaphoreType.DMA,
            pltpu.SemaphoreType.DMA,
        ],
    )
    def gather(table_hbm, idx_hbm, out_hbm, idx_v, buf_a, buf_b,
               gsem_a, gsem_b, wsem_a, wsem_b):
        wid = lax.axis_index("s") * NC + lax.axis_index("c")
        b0 = wid * bblk
        pltpu.sync_copy(idx_hbm.at[wid], idx_v)

        def wr(buf_, b, wsem_):
            pltpu.async_copy(
                buf_, out_hbm.at[b, pl.ds(0, S), pl.ds(0, D)], wsem_)

        def wait_write(buf_, wsem_):
            for _ in range(2):
                pltpu.make_async_copy(
                    buf_.at[pl.ds(0, S)],
                    out_hbm.at[0, pl.ds(0, S), pl.ds(0, D)], wsem_).wait()

        def wait_gather(buf_, gsem_):
            pltpu.make_async_copy(
                table_hbm.at[pl.ds(0, CH)], buf_, gsem_).wait()

        pltpu.async_copy(table_hbm.at[idx_v.at[0]], buf_a, gsem_a)

        def step(k, buf, gsem, wsem, obuf, ogsem, owsem):
            @pl.when(k + 1 < nch)
            def _():
                @pl.when(k >= 1)
                def _():
                    wait_write(obuf, owsem)

                pltpu.async_copy(table_hbm.at[idx_v.at[k + 1]], obuf, ogsem)

            wait_gather(buf, gsem)
            wr(buf.at[pl.ds(0, S)], b0 + 2 * k, wsem)
            wr(buf.at[pl.ds(S, S)], b0 + 2 * k + 1, wsem)

        def body(k, _):
            @pl.when(k % 2 == 0)
            def _():
                step(k, buf_a, gsem_a, wsem_a, buf_b, gsem_b, wsem_b)

            @pl.when(k % 2 == 1)
            def _():
                step(k, buf_b, gsem_b, wsem_b, buf_a, gsem_a, wsem_a)

            return 0

        lax.fori_loop(0, nch, body, 0)
        wait_write(buf_a, wsem_a)
        wait_write(buf_b, wsem_b)

    return gather


def kernel(inputs, embeddings, w):
    Bt, S = inputs.shape
    V, D = embeddings.shape
    B = Bt * S
    emb = _dense_mul_t(embeddings.T, w.T).reshape(2 * V, D)
    info = plsc.get_sparse_core_info()
    NW = info.num_cores * info.num_subcores
    ids3d = (inputs.reshape(NW, B // (NW * _CH), _CH) * 2).astype(jnp.int32)
    out56 = _make_gather(2 * V, D, B, Bt, S)(emb, ids3d)
    return out56[:, :S, :D]


# speedup vs baseline: 3.2691x; 1.0732x over previous
import functools

import jax
import jax.numpy as jnp
from jax import lax
from jax.experimental import pallas as pl
from jax.experimental.pallas import tpu as pltpu
from jax.experimental.pallas import tpu_sc as plsc


def _mul_t_body(e_ref, w_ref, o_ref):
    t = (e_ref[...] * w_ref[...]).T
    o_ref[:, : t.shape[1]] = t
    o_ref[:, t.shape[1]:] = jnp.zeros_like(t)


def _dense_mul_t(eT, wT):
    D_, V_ = eT.shape
    cols = 12800
    grid = (V_ + cols - 1) // cols
    in_spec = pl.BlockSpec((D_, cols), lambda i: (0, i))
    out_spec = pl.BlockSpec((cols, 2 * D_), lambda i: (i, 0))
    return pl.pallas_call(
        _mul_t_body,
        out_shape=jax.ShapeDtypeStruct((V_, 2 * D_), eT.dtype),
        grid=(grid,),
        in_specs=[in_spec, in_spec],
        out_specs=out_spec,
    )(eT, wT)


_CH = 100
_SPAD = 56


def _make_gather(V2, D, B, Bt, S):
    info = plsc.get_sparse_core_info()
    NC, NS = info.num_cores, info.num_subcores
    NW = NC * NS
    bpw = B // NW
    CH = _CH
    nch = bpw // CH
    bblk = Bt // NW
    assert B % NW == 0 and bpw % CH == 0 and CH == 2 * S

    mesh = plsc.VectorSubcoreMesh(core_axis_name="c", subcore_axis_name="s")

    @functools.partial(
        pl.kernel, mesh=mesh,
        out_type=jax.ShapeDtypeStruct((Bt, _SPAD, 2 * D), jnp.float32),
        compiler_params=pltpu.CompilerParams(use_tc_tiling_on_sc=False),
        scratch_types=[
            pltpu.VMEM((nch, CH), jnp.int32),
            pltpu.VMEM((CH, D), jnp.float32),
            pltpu.VMEM((CH, D), jnp.float32),
            pltpu.VMEM((CH, D), jnp.float32),
            pltpu.SemaphoreType.DMA,
            pltpu.SemaphoreType.DMA,
            pltpu.SemaphoreType.DMA,
            pltpu.SemaphoreType.DMA,
            pltpu.SemaphoreType.DMA,
            pltpu.SemaphoreType.DMA,
        ],
    )
    def gather(table_hbm, idx_hbm, out_hbm, idx_v, buf_a, buf_b, buf_c,
               gsem_a, gsem_b, gsem_c, wsem_a, wsem_b, wsem_c):
        wid = lax.axis_index("s") * NC + lax.axis_index("c")
        b0 = wid * bblk
        pltpu.sync_copy(idx_hbm.at[wid], idx_v)

        def wr(buf_, b, wsem_):
            pltpu.async_copy(
                buf_, out_hbm.at[b, pl.ds(0, S), pl.ds(0, D)], wsem_)

        def wait_write(buf_, wsem_):
            for _ in range(2):
                pltpu.make_async_copy(
                    buf_.at[pl.ds(0, S)],
                    out_hbm.at[0, pl.ds(0, S), pl.ds(0, D)], wsem_).wait()

        def wait_gather(buf_, gsem_):
            pltpu.make_async_copy(
                table_hbm.at[pl.ds(0, CH)], buf_, gsem_).wait()

        pltpu.async_copy(table_hbm.at[idx_v.at[0]], buf_a, gsem_a)
        pltpu.async_copy(table_hbm.at[idx_v.at[1]], buf_b, gsem_b)

        def step(k, buf, gsem, wsem, nbuf, ngsem, nwsem):
            @pl.when(k + 2 < nch)
            def _():
                @pl.when(k >= 1)
                def _():
                    wait_write(nbuf, nwsem)

                pltpu.async_copy(table_hbm.at[idx_v.at[k + 2]], nbuf, ngsem)

            wait_gather(buf, gsem)
            wr(buf.at[pl.ds(0, S)], b0 + 2 * k, wsem)
            wr(buf.at[pl.ds(S, S)], b0 + 2 * k + 1, wsem)

        def body(k, _):
            @pl.when(k % 3 == 0)
            def _():
                step(k, buf_a, gsem_a, wsem_a, buf_c, gsem_c, wsem_c)

            @pl.when(k % 3 == 1)
            def _():
                step(k, buf_b, gsem_b, wsem_b, buf_a, gsem_a, wsem_a)

            @pl.when(k % 3 == 2)
            def _():
                step(k, buf_c, gsem_c, wsem_c, buf_b, gsem_b, wsem_b)

            return 0

        lax.fori_loop(0, nch, body, 0)
        wait_write(buf_a, wsem_a)
        wait_write(buf_b, wsem_b)
        wait_write(buf_c, wsem_c)

    return gather


def kernel(inputs, embeddings, w):
    Bt, S = inputs.shape
    V, D = embeddings.shape
    B = Bt * S
    emb = _dense_mul_t(embeddings.T, w.T).reshape(2 * V, D)
    info = plsc.get_sparse_core_info()
    NW = info.num_cores * info.num_subcores
    ids3d = (inputs.reshape(NW, B // (NW * _CH), _CH) * 2).astype(jnp.int32)
    out56 = _make_gather(2 * V, D, B, Bt, S)(emb, ids3d)
    return out56[:, :S, :D]


# speedup vs baseline: 3.3181x; 1.0150x over previous
import functools

import jax
import jax.numpy as jnp
from jax import lax
from jax.experimental import pallas as pl
from jax.experimental.pallas import tpu as pltpu
from jax.experimental.pallas import tpu_sc as plsc


def _mul_t_body(e_ref, w_ref, o_ref):
    t = (e_ref[...] * w_ref[...]).T
    o_ref[:, : t.shape[1]] = t
    o_ref[:, t.shape[1]:] = jnp.zeros_like(t)


def _dense_mul_t(eT, wT):
    D_, V_ = eT.shape
    cols = 19200
    grid = (V_ + cols - 1) // cols
    in_spec = pl.BlockSpec((D_, cols), lambda i: (0, i))
    out_spec = pl.BlockSpec((cols, 2 * D_), lambda i: (i, 0))
    return pl.pallas_call(
        _mul_t_body,
        out_shape=jax.ShapeDtypeStruct((V_, 2 * D_), eT.dtype),
        grid=(grid,),
        in_specs=[in_spec, in_spec],
        out_specs=out_spec,
    )(eT, wT)


_CH = 100
_SPAD = 56


def _make_gather(V2, D, B, Bt, S):
    info = plsc.get_sparse_core_info()
    NC, NS = info.num_cores, info.num_subcores
    NW = NC * NS
    bpw = B // NW
    CH = _CH
    nch = bpw // CH
    bblk = Bt // NW
    assert B % NW == 0 and bpw % CH == 0 and CH == 2 * S

    mesh = plsc.VectorSubcoreMesh(core_axis_name="c", subcore_axis_name="s")

    @functools.partial(
        pl.kernel, mesh=mesh,
        out_type=jax.ShapeDtypeStruct((Bt, _SPAD, 2 * D), jnp.float32),
        compiler_params=pltpu.CompilerParams(use_tc_tiling_on_sc=False),
        scratch_types=[
            pltpu.VMEM((nch, CH), jnp.int32),
            pltpu.VMEM((CH, D), jnp.float32),
            pltpu.VMEM((CH, D), jnp.float32),
            pltpu.VMEM((CH, D), jnp.float32),
            pltpu.VMEM((CH, D), jnp.float32),
            pltpu.SemaphoreType.DMA,
            pltpu.SemaphoreType.DMA,
            pltpu.SemaphoreType.DMA,
            pltpu.SemaphoreType.DMA,
            pltpu.SemaphoreType.DMA,
            pltpu.SemaphoreType.DMA,
            pltpu.SemaphoreType.DMA,
            pltpu.SemaphoreType.DMA,
        ],
    )
    def gather(table_hbm, idx_hbm, out_hbm, idx_v, buf_a, buf_b, buf_c, buf_d,
               gsem_a, gsem_b, gsem_c, gsem_d, wsem_a, wsem_b, wsem_c, wsem_d):
        wid = lax.axis_index("s") * NC + lax.axis_index("c")
        b0 = wid * bblk
        pltpu.sync_copy(idx_hbm.at[wid], idx_v)

        def wr(buf_, b, wsem_):
            pltpu.async_copy(
                buf_, out_hbm.at[b, pl.ds(0, S), pl.ds(0, D)], wsem_)

        def wait_write(buf_, wsem_):
            for _ in range(2):
                pltpu.make_async_copy(
                    buf_.at[pl.ds(0, S)],
                    out_hbm.at[0, pl.ds(0, S), pl.ds(0, D)], wsem_).wait()

        def wait_gather(buf_, gsem_):
            pltpu.make_async_copy(
                table_hbm.at[pl.ds(0, CH)], buf_, gsem_).wait()

        pltpu.async_copy(table_hbm.at[idx_v.at[0]], buf_a, gsem_a)
        pltpu.async_copy(table_hbm.at[idx_v.at[1]], buf_b, gsem_b)
        pltpu.async_copy(table_hbm.at[idx_v.at[2]], buf_c, gsem_c)

        def step(k, buf, gsem, wsem, nbuf, ngsem, nwsem):
            @pl.when(k + 3 < nch)
            def _():
                @pl.when(k >= 1)
                def _():
                    wait_write(nbuf, nwsem)

                pltpu.async_copy(table_hbm.at[idx_v.at[k + 3]], nbuf, ngsem)

            wait_gather(buf, gsem)
            wr(buf.at[pl.ds(0, S)], b0 + 2 * k, wsem)
            wr(buf.at[pl.ds(S, S)], b0 + 2 * k + 1, wsem)

        def body(k, _):
            @pl.when(k % 4 == 0)
            def _():
                step(k, buf_a, gsem_a, wsem_a, buf_d, gsem_d, wsem_d)

            @pl.when(k % 4 == 1)
            def _():
                step(k, buf_b, gsem_b, wsem_b, buf_a, gsem_a, wsem_a)

            @pl.when(k % 4 == 2)
            def _():
                step(k, buf_c, gsem_c, wsem_c, buf_b, gsem_b, wsem_b)

            @pl.when(k % 4 == 3)
            def _():
                step(k, buf_d, gsem_d, wsem_d, buf_c, gsem_c, wsem_c)

            return 0

        lax.fori_loop(0, nch, body, 0)
        wait_write(buf_a, wsem_a)
        wait_write(buf_b, wsem_b)
        wait_write(buf_c, wsem_c)
        wait_write(buf_d, wsem_d)

    return gather


def kernel(inputs, embeddings, w):
    Bt, S = inputs.shape
    V, D = embeddings.shape
    B = Bt * S
    emb = _dense_mul_t(embeddings.T, w.T).reshape(2 * V, D)
    info = plsc.get_sparse_core_info()
    NW = info.num_cores * info.num_subcores
    ids3d = (inputs.reshape(NW, B // (NW * _CH), _CH) * 2).astype(jnp.int32)
    out56 = _make_gather(2 * V, D, B, Bt, S)(emb, ids3d)
    return out56[:, :S, :D]
